# baseline (speedup 1.0000x reference)
import functools

import jax
import jax.numpy as jnp
from jax import lax
from jax.experimental import pallas as pl
from jax.experimental.pallas import tpu as pltpu

N, C, D, H, W = 1, 128, 16, 32, 32
G, K, PAD = 2, 3, 1
P = K ** 3
Cg = C // G
M = D * H * W
Dp, Hp, Wp = D + 2, H + 2, W + 2
R = Dp * Hp * Wp
RT = R + 8
TAB = G * RT
E = P * 8
EP = 224
SHIFT_PAD = D * H * W // D + W + 1
_SHIFT0 = H * W + W + 1


def _k1a_body(x_ref, prew_ref, pre_ref):
    pre_ref[...] = lax.dot_general(x_ref[...], prew_ref[...],
                                   (((1,), (1,)), ((), ())),
                                   preferred_element_type=jnp.float32)


_HW = H * W
_SH = W + 1


def _k1_body(prev_ref, cur_ref, next_ref, dwk_ref, dw_ref):
    row = lax.broadcasted_iota(jnp.int32, (_HW, C), 0)
    h = row // W
    w = row % W
    acc = jnp.zeros((_HW, C), jnp.float32)
    z = jnp.zeros((_SH, C), jnp.float32)
    for kd, src in enumerate((prev_ref, cur_ref, next_ref)):
        plane = jnp.concatenate([z, src[...], z], axis=0)
        for kh in range(K):
            for kw in range(K):
                s = (kh - 1) * W + (kw - 1)
                win = lax.slice(plane, (_SH + s, 0), (_SH + s + _HW, C))
                vh = (h + (kh - 1) >= 0) & (h + (kh - 1) < H)
                vw = (w + (kw - 1) >= 0) & (w + (kw - 1) < W)
                k = kd * 9 + kh * 3 + kw
                acc = acc + jnp.where(vh & vw, win, 0.0) * dwk_ref[k, :][None, :]
    dw_ref[...] = acc


def _k1c_body(dw_ref, dwb_ref, feat_ref):
    acc = dw_ref[...] + dwb_ref[0, :][None, :]
    mu = jnp.mean(acc, axis=0, keepdims=True)
    cen = acc - mu
    var = jnp.mean(cen * cen, axis=0, keepdims=True)
    dwn = cen / jnp.sqrt(var + 1e-5)
    feat_ref[...] = jax.nn.gelu(dwn)


def _k2_body(pre_ref, feat_ref, ipw_ref, ipb_ref,
             wd0_ref, wd1_ref, wh0_ref, wh1_ref, ww0_ref, ww1_ref,
             bd_ref, bh_ref, bw_ref, mk0_ref, mk1_ref, mb_ref,
             xproj_ref, od0_ref, od1_ref, oh0_ref, oh1_ref,
             ow0_ref, ow1_ref, ms0_ref, ms1_ref):
    pre = pre_ref[...]
    feat = feat_ref[...]
    xproj_ref[...] = (jnp.dot(pre, ipw_ref[...],
                              preferred_element_type=jnp.float32)
                      + ipb_ref[0, :][None, :])

    def proj(wref, brow):
        return (jnp.dot(feat, wref[...], preferred_element_type=jnp.float32)
                + brow[None, :])

    od0_ref[...] = proj(wd0_ref, bd_ref[0, :P])
    od1_ref[...] = proj(wd1_ref, bd_ref[0, P:])
    oh0_ref[...] = proj(wh0_ref, bh_ref[0, :P])
    oh1_ref[...] = proj(wh1_ref, bh_ref[0, P:])
    ow0_ref[...] = proj(ww0_ref, bw_ref[0, :P])
    ow1_ref[...] = proj(ww1_ref, bw_ref[0, P:])

    def smax(wref, brow):
        l = (jnp.dot(feat, wref[...], preferred_element_type=jnp.float32)
             + brow[None, :])
        l = l - jnp.max(l, axis=1, keepdims=True)
        e = jnp.exp(l)
        return e / jnp.sum(e, axis=1, keepdims=True)

    ms0_ref[...] = smax(mk0_ref, mb_ref[0, :P])
    ms1_ref[...] = smax(mk1_ref, mb_ref[0, P:])


def _k3_body(od0_ref, od1_ref, oh0_ref, oh1_ref, ow0_ref, ow1_ref,
             ms0_ref, ms1_ref, kd_ref, kh_ref, kw_ref,
             idx0_ref, idx1_ref, wgt0_ref, wgt1_ref):
    pid = pl.program_id(0)
    bm = od0_ref.shape[0]
    row = pid * bm + lax.broadcasted_iota(jnp.int32, (bm, P), 0)
    based = (row // (H * W)).astype(jnp.float32)
    baseh = ((row // W) % H).astype(jnp.float32)
    basew = (row % W).astype(jnp.float32)
    kdv = kd_ref[0, :][None, :]
    khv = kh_ref[0, :][None, :]
    kwv = kw_ref[0, :][None, :]

    for g, (odr, ohr, owr, msr, iref, wref) in enumerate(
            ((od0_ref, oh0_ref, ow0_ref, ms0_ref, idx0_ref, wgt0_ref),
             (od1_ref, oh1_ref, ow1_ref, ms1_ref, idx1_ref, wgt1_ref))):
        cd = jnp.clip(based + kdv + odr[...], -4.0, 40.0)
        ch = jnp.clip(baseh + khv + ohr[...], -4.0, 40.0)
        cw = jnp.clip(basew + kwv + owr[...], -4.0, 40.0)
        fd = jnp.floor(cd)
        fh = jnp.floor(ch)
        fw = jnp.floor(cw)
        rd = cd - fd
        rh = ch - fh
        rw = cw - fw
        i0d = fd.astype(jnp.int32)
        i0h = fh.astype(jnp.int32)
        i0w = fw.astype(jnp.int32)
        msk = msr[...]
        goff = g * RT
        ilist, wlist = [], []
        for dd in range(2):
            idd = i0d + dd
            vd = (idd >= 0) & (idd < Dp)
            wd = rd if dd else 1.0 - rd
            for dh in range(2):
                idh = i0h + dh
                vh = (idh >= 0) & (idh < Hp)
                wh = rh if dh else 1.0 - rh
                for dw_ in range(2):
                    idw = i0w + dw_
                    vw = (idw >= 0) & (idw < Wp)
                    ww = rw if dw_ else 1.0 - rw
                    valid = vd & vh & vw
                    flat = idd * (Hp * Wp) + idh * Wp + idw
                    ilist.append(jnp.where(valid, flat, R) + goff)
                    wlist.append(jnp.where(valid, wd * wh * ww * msk, 0.0))
        ilist.append(jnp.zeros((bm, EP - E), jnp.int32))
        wlist.append(jnp.zeros((bm, EP - E), jnp.float32))
        iref[...] = jnp.concatenate(ilist, axis=1)
        wref[...] = jnp.concatenate(wlist, axis=1)


def _k4_body(x_ref, out_ref, opw_ref, opb_ref, postw_ref, gate_ref, res_ref):
    out2 = (jnp.dot(out_ref[...], opw_ref[...],
                    preferred_element_type=jnp.float32)
            + opb_ref[0, :][None, :])
    y = lax.dot_general(out2, postw_ref[...], (((1,), (1,)), ((), ())),
                        preferred_element_type=jnp.float32)
    sig = 1.0 / (1.0 + jnp.exp(-gate_ref[0]))
    res_ref[...] = x_ref[...] + sig * y


def _f32(shape):
    return jax.ShapeDtypeStruct(shape, jnp.float32)


def _i32(shape):
    return jax.ShapeDtypeStruct(shape, jnp.int32)


def kernel(x, pre_w, dw_w, dw_b, ip_w, ip_b, off_w, off_b, mk_w, mk_b,
           op_w, op_b, post_w, gate):
    x_cl = x.reshape(C, M).T

    dwk = dw_w.reshape(C, P).T
    ow4 = off_w.reshape(C, G, P, 3)
    ob4 = off_b.reshape(G, P, 3)
    wd0, wd1 = ow4[:, 0, :, 2], ow4[:, 1, :, 2]
    wh0, wh1 = ow4[:, 0, :, 1], ow4[:, 1, :, 1]
    ww0, ww1 = ow4[:, 0, :, 0], ow4[:, 1, :, 0]
    bd = ob4[:, :, 2].reshape(1, G * P)
    bh = ob4[:, :, 1].reshape(1, G * P)
    bw = ob4[:, :, 0].reshape(1, G * P)
    mk0, mk1 = mk_w[:, :P], mk_w[:, P:]

    NB, BM = 8, M // 8
    mspec = pl.BlockSpec((BM, C), lambda i: (i, 0))
    pspec = pl.BlockSpec((BM, P), lambda i: (i, 0))
    wspec = pl.BlockSpec((C, C), lambda i: (0, 0))
    w27 = pl.BlockSpec((C, P), lambda i: (0, 0))
    b1c = pl.BlockSpec((1, C), lambda i: (0, 0))
    b154 = pl.BlockSpec((1, G * P), lambda i: (0, 0))

    pre_cl = pl.pallas_call(
        _k1a_body,
        grid=(NB,),
        in_specs=[mspec, wspec],
        out_specs=mspec,
        out_shape=_f32((M, C)),
    )(x_cl, pre_w)

    pre_prev = jnp.pad(pre_cl, ((_HW, 0), (0, 0)))[:M]
    pre_next = jnp.pad(pre_cl, ((0, _HW), (0, 0)))[_HW:]
    dspec = pl.BlockSpec((_HW, C), lambda i: (i, 0))
    dwcv = pl.pallas_call(
        _k1_body,
        grid=(D,),
        in_specs=[dspec, dspec, dspec, pl.BlockSpec((P, C), lambda i: (0, 0))],
        out_specs=dspec,
        out_shape=_f32((M, C)),
    )(pre_prev, pre_cl, pre_next, dwk)
    feat = pl.pallas_call(
        _k1c_body,
        out_shape=_f32((M, C)),
    )(dwcv, dw_b.reshape(1, C))

    (x_proj, od0, od1, oh0, oh1, ow0, ow1, ms0, ms1) = pl.pallas_call(
        _k2_body,
        grid=(NB,),
        in_specs=[mspec, mspec, wspec, b1c] + [w27] * 6 + [b154] * 3
        + [w27] * 2 + [b154],
        out_specs=[mspec] + [pspec] * 8,
        out_shape=[_f32((M, C))] + [_f32((M, P))] * 8,
    )(pre_cl, feat, ip_w, ip_b.reshape(1, C),
      wd0, wd1, wh0, wh1, ww0, ww1, bd, bh, bw, mk0, mk1,
      mk_b.reshape(1, G * P))

    NB3, BM3 = 16, M // 16
    bspec = pl.BlockSpec((BM3, P), lambda i: (i, 0))
    kspec = pl.BlockSpec((1, P), lambda i: (0, 0))
    espec = pl.BlockSpec((BM3, EP), lambda i: (i, 0))
    kdv = jnp.repeat(jnp.arange(K, dtype=jnp.float32), K * K).reshape(1, P)
    khv = jnp.tile(jnp.repeat(jnp.arange(K, dtype=jnp.float32), K),
                   K).reshape(1, P)
    kwv = jnp.tile(jnp.arange(K, dtype=jnp.float32), K * K).reshape(1, P)
    idx0, idx1, wgt0, wgt1 = pl.pallas_call(
        _k3_body,
        grid=(NB3,),
        in_specs=[bspec] * 8 + [kspec] * 3,
        out_specs=[espec] * 4,
        out_shape=[_i32((M, EP)), _i32((M, EP)),
                   _f32((M, EP)), _f32((M, EP))],
    )(od0, od1, oh0, oh1, ow0, ow1, ms0, ms1, kdv, khv, kwv)

    xp = jnp.pad(x_proj.reshape(D, H, W, C),
                 ((1, 1), (1, 1), (1, 1), (0, 0)))
    tab = xp.reshape(R, G, Cg).transpose(1, 0, 2)
    tab = jnp.pad(tab, ((0, 0), (0, RT - R), (0, 0))).reshape(TAB, Cg)

    idx = jnp.concatenate([idx0, idx1], axis=0)
    wgt = jnp.concatenate([wgt0, wgt1], axis=0)

    out = jnp.einsum('bec,be->bc', tab[idx], wgt,
                     preferred_element_type=jnp.float32)

    out_cl = out.reshape(G, M, Cg).transpose(1, 0, 2).reshape(M, C)
    res = pl.pallas_call(
        _k4_body,
        grid=(NB,),
        in_specs=[mspec, mspec, wspec, b1c, wspec,
                  pl.BlockSpec(memory_space=pltpu.SMEM)],
        out_specs=mspec,
        out_shape=_f32((M, C)),
    )(x_cl, out_cl, op_w, op_b.reshape(1, C), post_w, gate.reshape(1))
    return res.T.reshape(N, C, D, H, W)


# speedup vs baseline: 4.3978x; 4.3978x over previous
import functools

import jax
import jax.numpy as jnp
from jax import lax
from jax.experimental import pallas as pl
from jax.experimental.pallas import tpu as pltpu
from jax.experimental.pallas import tpu_sc as plsc

N, C, D, H, W = 1, 128, 16, 32, 32
G, K, PAD = 2, 3, 1
P = K ** 3
Cg = C // G
M = D * H * W
Dp, Hp, Wp = D + 2, H + 2, W + 2
R = Dp * Hp * Wp
RT = R + 8
TAB = G * RT
E = P * 8
EP = 224
SHIFT_PAD = D * H * W // D + W + 1
_SHIFT0 = H * W + W + 1


def _k1a_body(x_ref, prew_ref, pre_ref):
    pre_ref[...] = lax.dot_general(x_ref[...], prew_ref[...],
                                   (((1,), (1,)), ((), ())),
                                   preferred_element_type=jnp.float32)


_HW = H * W
_SH = W + 1


def _k1_body(prev_ref, cur_ref, next_ref, dwk_ref, dw_ref):
    row = lax.broadcasted_iota(jnp.int32, (_HW, C), 0)
    h = row // W
    w = row % W
    acc = jnp.zeros((_HW, C), jnp.float32)
    z = jnp.zeros((_SH, C), jnp.float32)
    for kd, src in enumerate((prev_ref, cur_ref, next_ref)):
        plane = jnp.concatenate([z, src[...], z], axis=0)
        for kh in range(K):
            for kw in range(K):
                s = (kh - 1) * W + (kw - 1)
                win = lax.slice(plane, (_SH + s, 0), (_SH + s + _HW, C))
                vh = (h + (kh - 1) >= 0) & (h + (kh - 1) < H)
                vw = (w + (kw - 1) >= 0) & (w + (kw - 1) < W)
                k = kd * 9 + kh * 3 + kw
                acc = acc + jnp.where(vh & vw, win, 0.0) * dwk_ref[k, :][None, :]
    dw_ref[...] = acc


def _k1c_body(dw_ref, dwb_ref, feat_ref):
    acc = dw_ref[...] + dwb_ref[0, :][None, :]
    mu = jnp.mean(acc, axis=0, keepdims=True)
    cen = acc - mu
    var = jnp.mean(cen * cen, axis=0, keepdims=True)
    dwn = cen / jnp.sqrt(var + 1e-5)
    feat_ref[...] = jax.nn.gelu(dwn)


def _k2_body(pre_ref, feat_ref, ipw_ref, ipb_ref,
             wd0_ref, wd1_ref, wh0_ref, wh1_ref, ww0_ref, ww1_ref,
             bd_ref, bh_ref, bw_ref, mk0_ref, mk1_ref, mb_ref,
             xproj_ref, od0_ref, od1_ref, oh0_ref, oh1_ref,
             ow0_ref, ow1_ref, ms0_ref, ms1_ref):
    pre = pre_ref[...]
    feat = feat_ref[...]
    xproj_ref[...] = (jnp.dot(pre, ipw_ref[...],
                              preferred_element_type=jnp.float32)
                      + ipb_ref[0, :][None, :])

    def proj(wref, brow):
        return (jnp.dot(feat, wref[...], preferred_element_type=jnp.float32)
                + brow[None, :])

    od0_ref[...] = proj(wd0_ref, bd_ref[0, :P])
    od1_ref[...] = proj(wd1_ref, bd_ref[0, P:])
    oh0_ref[...] = proj(wh0_ref, bh_ref[0, :P])
    oh1_ref[...] = proj(wh1_ref, bh_ref[0, P:])
    ow0_ref[...] = proj(ww0_ref, bw_ref[0, :P])
    ow1_ref[...] = proj(ww1_ref, bw_ref[0, P:])

    def smax(wref, brow):
        l = (jnp.dot(feat, wref[...], preferred_element_type=jnp.float32)
             + brow[None, :])
        l = l - jnp.max(l, axis=1, keepdims=True)
        e = jnp.exp(l)
        return e / jnp.sum(e, axis=1, keepdims=True)

    ms0_ref[...] = smax(mk0_ref, mb_ref[0, :P])
    ms1_ref[...] = smax(mk1_ref, mb_ref[0, P:])


def _k3_body(od0_ref, od1_ref, oh0_ref, oh1_ref, ow0_ref, ow1_ref,
             ms0_ref, ms1_ref, kd_ref, kh_ref, kw_ref,
             idx0_ref, idx1_ref, wgt0_ref, wgt1_ref):
    pid = pl.program_id(0)
    bm = od0_ref.shape[0]
    row = pid * bm + lax.broadcasted_iota(jnp.int32, (bm, P), 0)
    based = (row // (H * W)).astype(jnp.float32)
    baseh = ((row // W) % H).astype(jnp.float32)
    basew = (row % W).astype(jnp.float32)
    kdv = kd_ref[0, :][None, :]
    khv = kh_ref[0, :][None, :]
    kwv = kw_ref[0, :][None, :]

    for g, (odr, ohr, owr, msr, iref, wref) in enumerate(
            ((od0_ref, oh0_ref, ow0_ref, ms0_ref, idx0_ref, wgt0_ref),
             (od1_ref, oh1_ref, ow1_ref, ms1_ref, idx1_ref, wgt1_ref))):
        cd = jnp.clip(based + kdv + odr[...], -4.0, 40.0)
        ch = jnp.clip(baseh + khv + ohr[...], -4.0, 40.0)
        cw = jnp.clip(basew + kwv + owr[...], -4.0, 40.0)
        fd = jnp.floor(cd)
        fh = jnp.floor(ch)
        fw = jnp.floor(cw)
        rd = cd - fd
        rh = ch - fh
        rw = cw - fw
        i0d = fd.astype(jnp.int32)
        i0h = fh.astype(jnp.int32)
        i0w = fw.astype(jnp.int32)
        msk = msr[...]
        goff = g * RT
        ilist, wlist = [], []
        for dd in range(2):
            idd = i0d + dd
            vd = (idd >= 0) & (idd < Dp)
            wd = rd if dd else 1.0 - rd
            for dh in range(2):
                idh = i0h + dh
                vh = (idh >= 0) & (idh < Hp)
                wh = rh if dh else 1.0 - rh
                for dw_ in range(2):
                    idw = i0w + dw_
                    vw = (idw >= 0) & (idw < Wp)
                    ww = rw if dw_ else 1.0 - rw
                    valid = vd & vh & vw
                    flat = idd * (Hp * Wp) + idh * Wp + idw
                    ilist.append(jnp.where(valid, flat, R) + goff)
                    wlist.append(jnp.where(valid, wd * wh * ww * msk, 0.0))
        ilist.append(jnp.zeros((bm, EP - E), jnp.int32))
        wlist.append(jnp.zeros((bm, EP - E), jnp.float32))
        iref[...] = jnp.concatenate(ilist, axis=1)
        wref[...] = jnp.concatenate(wlist, axis=1)


EH = EP // 2
NC, NS = 2, 16
NW = NC * NS
SB = 16


def _sc_sample_body(tab_hbm, idx_hbm, wgt_hbm, out_hbm,
                    idx_v, wgt_v, rows_v, outb_v, sem0, sem1):
    wid = lax.axis_index("s") * NC + lax.axis_index("c")
    vpt = G * M // NW
    row0 = wid * vpt

    def batch_body(bi, carry):
        r0 = row0 + bi * SB
        pltpu.sync_copy(idx_hbm.at[pl.ds(r0, SB)], idx_v)
        pltpu.sync_copy(wgt_hbm.at[pl.ds(r0, SB)], wgt_v)
        for b in range(SB):
            cp0 = pltpu.async_copy(tab_hbm.at[idx_v.at[b, 0]],
                                   rows_v.at[pl.ds(0, EH)], sem0)
            cp1 = pltpu.async_copy(tab_hbm.at[idx_v.at[b, 1]],
                                   rows_v.at[pl.ds(EH, EH)], sem1)
            cp0.wait()
            cp1.wait()

            def jo_body(jo, acc):
                j0 = jo * 16
                w16 = wgt_v[b, pl.ds(j0, 16)]
                acc = list(acc)
                for lane in range(16):
                    w = w16[lane]
                    for c in range(4):
                        acc[c] = acc[c] + w * rows_v[j0 + lane,
                                                     pl.ds(c * 16, 16)]
                return tuple(acc)

            acc = lax.fori_loop(0, EP // 16, jo_body,
                                tuple(jnp.zeros((16,), jnp.float32)
                                      for _ in range(4)))
            for c in range(4):
                outb_v[b, pl.ds(c * 16, 16)] = acc[c]
        pltpu.sync_copy(outb_v, out_hbm.at[pl.ds(r0, SB)])
        return carry

    lax.fori_loop(0, (G * M) // NW // SB, batch_body, 0)


def _sc_sample(tab, idx, wgt):
    mesh = plsc.VectorSubcoreMesh(core_axis_name="c", subcore_axis_name="s",
                                  num_cores=NC, num_subcores=NS)
    fn = functools.partial(
        pl.kernel, mesh=mesh,
        compiler_params=pltpu.CompilerParams(use_tc_tiling_on_sc=False),
        out_type=jax.ShapeDtypeStruct((G * M, Cg), jnp.float32),
        scratch_types=[
            pltpu.VMEM((SB, 2, EH), jnp.int32),
            pltpu.VMEM((SB, EP), jnp.float32),
            pltpu.VMEM((EP, Cg), jnp.float32),
            pltpu.VMEM((SB, Cg), jnp.float32),
            pltpu.SemaphoreType.DMA,
            pltpu.SemaphoreType.DMA,
        ],
    )(_sc_sample_body)
    return fn(tab, idx.reshape(G * M, 2, EH), wgt)


def _k4_body(x_ref, out_ref, opw_ref, opb_ref, postw_ref, gate_ref, res_ref):
    out2 = (jnp.dot(out_ref[...], opw_ref[...],
                    preferred_element_type=jnp.float32)
            + opb_ref[0, :][None, :])
    y = lax.dot_general(out2, postw_ref[...], (((1,), (1,)), ((), ())),
                        preferred_element_type=jnp.float32)
    sig = 1.0 / (1.0 + jnp.exp(-gate_ref[0]))
    res_ref[...] = x_ref[...] + sig * y


def _f32(shape):
    return jax.ShapeDtypeStruct(shape, jnp.float32)


def _i32(shape):
    return jax.ShapeDtypeStruct(shape, jnp.int32)


def kernel(x, pre_w, dw_w, dw_b, ip_w, ip_b, off_w, off_b, mk_w, mk_b,
           op_w, op_b, post_w, gate):
    x_cl = x.reshape(C, M).T

    dwk = dw_w.reshape(C, P).T
    ow4 = off_w.reshape(C, G, P, 3)
    ob4 = off_b.reshape(G, P, 3)
    wd0, wd1 = ow4[:, 0, :, 2], ow4[:, 1, :, 2]
    wh0, wh1 = ow4[:, 0, :, 1], ow4[:, 1, :, 1]
    ww0, ww1 = ow4[:, 0, :, 0], ow4[:, 1, :, 0]
    bd = ob4[:, :, 2].reshape(1, G * P)
    bh = ob4[:, :, 1].reshape(1, G * P)
    bw = ob4[:, :, 0].reshape(1, G * P)
    mk0, mk1 = mk_w[:, :P], mk_w[:, P:]

    NB, BM = 8, M // 8
    mspec = pl.BlockSpec((BM, C), lambda i: (i, 0))
    pspec = pl.BlockSpec((BM, P), lambda i: (i, 0))
    wspec = pl.BlockSpec((C, C), lambda i: (0, 0))
    w27 = pl.BlockSpec((C, P), lambda i: (0, 0))
    b1c = pl.BlockSpec((1, C), lambda i: (0, 0))
    b154 = pl.BlockSpec((1, G * P), lambda i: (0, 0))

    pre_cl = pl.pallas_call(
        _k1a_body,
        grid=(NB,),
        in_specs=[mspec, wspec],
        out_specs=mspec,
        out_shape=_f32((M, C)),
    )(x_cl, pre_w)

    pre_prev = jnp.pad(pre_cl, ((_HW, 0), (0, 0)))[:M]
    pre_next = jnp.pad(pre_cl, ((0, _HW), (0, 0)))[_HW:]
    dspec = pl.BlockSpec((_HW, C), lambda i: (i, 0))
    dwcv = pl.pallas_call(
        _k1_body,
        grid=(D,),
        in_specs=[dspec, dspec, dspec, pl.BlockSpec((P, C), lambda i: (0, 0))],
        out_specs=dspec,
        out_shape=_f32((M, C)),
    )(pre_prev, pre_cl, pre_next, dwk)
    feat = pl.pallas_call(
        _k1c_body,
        out_shape=_f32((M, C)),
    )(dwcv, dw_b.reshape(1, C))

    (x_proj, od0, od1, oh0, oh1, ow0, ow1, ms0, ms1) = pl.pallas_call(
        _k2_body,
        grid=(NB,),
        in_specs=[mspec, mspec, wspec, b1c] + [w27] * 6 + [b154] * 3
        + [w27] * 2 + [b154],
        out_specs=[mspec] + [pspec] * 8,
        out_shape=[_f32((M, C))] + [_f32((M, P))] * 8,
    )(pre_cl, feat, ip_w, ip_b.reshape(1, C),
      wd0, wd1, wh0, wh1, ww0, ww1, bd, bh, bw, mk0, mk1,
      mk_b.reshape(1, G * P))

    NB3, BM3 = 16, M // 16
    bspec = pl.BlockSpec((BM3, P), lambda i: (i, 0))
    kspec = pl.BlockSpec((1, P), lambda i: (0, 0))
    espec = pl.BlockSpec((BM3, EP), lambda i: (i, 0))
    kdv = jnp.repeat(jnp.arange(K, dtype=jnp.float32), K * K).reshape(1, P)
    khv = jnp.tile(jnp.repeat(jnp.arange(K, dtype=jnp.float32), K),
                   K).reshape(1, P)
    kwv = jnp.tile(jnp.arange(K, dtype=jnp.float32), K * K).reshape(1, P)
    idx0, idx1, wgt0, wgt1 = pl.pallas_call(
        _k3_body,
        grid=(NB3,),
        in_specs=[bspec] * 8 + [kspec] * 3,
        out_specs=[espec] * 4,
        out_shape=[_i32((M, EP)), _i32((M, EP)),
                   _f32((M, EP)), _f32((M, EP))],
    )(od0, od1, oh0, oh1, ow0, ow1, ms0, ms1, kdv, khv, kwv)

    xp = jnp.pad(x_proj.reshape(D, H, W, C),
                 ((1, 1), (1, 1), (1, 1), (0, 0)))
    tab = xp.reshape(R, G, Cg).transpose(1, 0, 2)
    tab = jnp.pad(tab, ((0, 0), (0, RT - R), (0, 0))).reshape(TAB, Cg)

    idx = jnp.concatenate([idx0, idx1], axis=0)
    wgt = jnp.concatenate([wgt0, wgt1], axis=0)

    out = _sc_sample(tab, idx, wgt)

    out_cl = out.reshape(G, M, Cg).transpose(1, 0, 2).reshape(M, C)
    res = pl.pallas_call(
        _k4_body,
        grid=(NB,),
        in_specs=[mspec, mspec, wspec, b1c, wspec,
                  pl.BlockSpec(memory_space=pltpu.SMEM)],
        out_specs=mspec,
        out_shape=_f32((M, C)),
    )(x_cl, out_cl, op_w, op_b.reshape(1, C), post_w, gate.reshape(1))
    return res.T.reshape(N, C, D, H, W)
